# baseline (speedup 1.0000x reference)
import functools

import jax
import jax.numpy as jnp
from jax import lax
from jax.experimental import pallas as pl
from jax.experimental.pallas import tpu as pltpu
from jax.experimental.pallas import tpu_sc as plsc

NC = 2
NS = 16
LANES = 16
NW = NC * NS
C = 80


def _matmul_stats_body(x_ref, w_ref, xw_ref, stats_ref):
    i = pl.program_id(0)
    xb = x_ref[...]
    r_count = w_ref.shape[0]
    for r in range(r_count):
        xw_ref[r] = jnp.dot(xb, w_ref[r], preferred_element_type=jnp.float32)
    s1 = jnp.sum(xb, axis=0, keepdims=True)
    s2 = jnp.sum(xb * xb, axis=0, keepdims=True)
    blk = jnp.concatenate([s1, s2], axis=0)

    @pl.when(i == 0)
    def _():
        stats_ref[...] = blk

    @pl.when(i > 0)
    def _():
        stats_ref[...] = stats_ref[...] + blk


def _tc_matmul_stats(x, W, bn):
    N, D = x.shape
    R = W.shape[0]
    nb = N // bn
    return pl.pallas_call(
        _matmul_stats_body,
        grid=(nb,),
        in_specs=[
            pl.BlockSpec((bn, D), lambda i: (i, 0)),
            pl.BlockSpec((R, D, D), lambda i: (0, 0, 0)),
        ],
        out_specs=[
            pl.BlockSpec((R, bn, D), lambda i: (0, i, 0)),
            pl.BlockSpec((2, D), lambda i: (0, 0)),
        ],
        out_shape=[
            jax.ShapeDtypeStruct((R, N, D), jnp.float32),
            jax.ShapeDtypeStruct((2, D), jnp.float32),
        ],
    )(x, W)


def _combine_body(n_rows, x_ref, part_ref, stats_ref, gamma_ref, beta_ref,
                  out_ref):
    st = stats_ref[...]
    inv_n = jnp.float32(1.0 / n_rows)
    mean = st[0:1] * inv_n
    var = st[1:2] * inv_n - mean * mean
    inv = lax.rsqrt(var + jnp.float32(1e-5))
    p = part_ref[0] + part_ref[1]
    h = (x_ref[...] - mean) * (inv * gamma_ref[...]) + beta_ref[...] + p
    out_ref[...] = jnp.maximum(h, jnp.float32(0.0))


def _tc_combine(x, partials, stats, gamma, beta, bn):
    N, D = x.shape
    nb = N // bn
    return pl.pallas_call(
        functools.partial(_combine_body, N),
        grid=(nb,),
        in_specs=[
            pl.BlockSpec((bn, D), lambda i: (i, 0)),
            pl.BlockSpec((2, bn, D), lambda i: (0, i, 0)),
            pl.BlockSpec((2, D), lambda i: (0, 0)),
            pl.BlockSpec((1, D), lambda i: (0, 0)),
            pl.BlockSpec((1, D), lambda i: (0, 0)),
        ],
        out_specs=pl.BlockSpec((bn, D), lambda i: (i, 0)),
        out_shape=jax.ShapeDtypeStruct((N, D), jnp.float32),
    )(x, partials, stats, gamma, beta)


NBUF = 4


def _sc_edges(table, edata, zeros, N, NPAD, D, G):
    rows_per_tile = NPAD // NS
    mesh = plsc.VectorSubcoreMesh(core_axis_name="c", subcore_axis_name="s")
    assert G % NBUF == 0 and G > 2 * NBUF

    @functools.partial(
        pl.kernel,
        mesh=mesh,
        out_type=jax.ShapeDtypeStruct((NC, NPAD, D), jnp.float32),
        scratch_types=[
            [pltpu.VMEM((4, C), jnp.float32) for _ in range(NBUF)],
            [pltpu.VMEM((C,), jnp.int32) for _ in range(NBUF)],
            [pltpu.VMEM((C,), jnp.int32) for _ in range(NBUF)],
            [pltpu.VMEM((C,), jnp.float32) for _ in range(NBUF)],
            [pltpu.VMEM((C, D), jnp.float32) for _ in range(NBUF)],
            [pltpu.SemaphoreType.DMA for _ in range(NBUF)],
            [pltpu.SemaphoreType.DMA for _ in range(NBUF)],
            [pltpu.SemaphoreType.DMA for _ in range(NBUF)],
            pltpu.VMEM_SHARED((NPAD, D), jnp.float32),
        ],
    )
    def k(table_hbm, edata_hbm, zeros_hbm, out_hbm,
          ebuf, idx_v, dst_v, norm_v, rows_v, sed, sg, ss, acc):
        cid = lax.axis_index("c")
        sid = lax.axis_index("s")
        wid = cid * NS + sid

        if True:
            zbase = sid * rows_per_tile
            pltpu.sync_copy(zeros_hbm.at[pl.ds(zbase, rows_per_tile)],
                            acc.at[pl.ds(zbase, rows_per_tile)])
            plsc.subcore_barrier()

            def issue_ed(i, b):
                pltpu.async_copy(edata_hbm.at[wid, i], ebuf[b], sed[b])

            def wait_ed(i, b):
                pltpu.make_async_copy(edata_hbm.at[wid, i], ebuf[b],
                                      sed[b]).wait()

            def issue_gather(b):
                pltpu.async_copy(table_hbm.at[idx_v[b]], rows_v[b], sg[b])

            def wait_gather(b):
                pltpu.make_async_copy(table_hbm.at[idx_v[b]], rows_v[b],
                                      sg[b]).wait()

            def issue_scatter(b):
                pltpu.async_copy(rows_v[b], acc.at[dst_v[b]], ss[b], add=True)

            def wait_scatter(b):
                pltpu.make_async_copy(rows_v[b], acc.at[dst_v[b]],
                                      ss[b]).wait()

            def compute_idx(b):
                for j in range(C // LANES):
                    sl = pl.ds(j * LANES, LANES)
                    sv = ebuf[b][0, sl]
                    rv = ebuf[b][1, sl]
                    idx_v[b][sl] = (rv * jnp.float32(N) + sv).astype(jnp.int32)
                    dst_v[b][sl] = ebuf[b][2, sl].astype(jnp.int32)
                    norm_v[b][sl] = ebuf[b][3, sl]

            def scale(b):
                def body(j, c2):
                    base = j * LANES
                    norm16 = norm_v[b][pl.ds(base, LANES)]
                    for t in range(LANES):
                        nb = norm16[t]
                        for v in range(D // LANES):
                            sl = pl.ds(v * LANES, LANES)
                            rows_v[b][base + t, sl] = (
                                rows_v[b][base + t, sl] * nb)
                    return c2

                lax.fori_loop(0, C // LANES, body, 0)

            for i in range(NBUF):
                issue_ed(i, i)
            wait_ed(0, 0)
            compute_idx(0)
            issue_ed(NBUF, 0)
            issue_gather(0)

            def outer(g5, carry):
                for b in range(NBUF):
                    bn = (b + 1) % NBUF
                    ii = g5 * NBUF + b
                    nxt = ii + 1

                    @pl.when(nxt < G)
                    def _():
                        @pl.when(nxt >= NBUF)
                        def _():
                            wait_scatter(bn)
                        wait_ed(nxt, bn)
                        compute_idx(bn)
                        issue_gather(bn)

                    @pl.when(nxt + NBUF < G)
                    def _():
                        issue_ed(nxt + NBUF, bn)

                    wait_gather(b)
                    scale(b)
                    issue_scatter(b)
                return carry

            lax.fori_loop(0, G // NBUF, outer, 0)
            for b in range(NBUF):
                wait_scatter(b)
            plsc.subcore_barrier()
            pltpu.sync_copy(acc.at[pl.ds(zbase, rows_per_tile)],
                            out_hbm.at[cid, pl.ds(zbase, rows_per_tile)])

    return k(table, edata, zeros)


def kernel(x, edge_index, rel_type, norm, W, gamma, beta):
    N, D = x.shape
    E = rel_type.shape[0]
    R = W.shape[0]
    G = -(-E // (NW * C))
    G = -(-G // NBUF) * NBUF
    e_pad = NW * C * G - E
    NPAD = -(-N // (NS * 8)) * (NS * 8)

    xw, stats = _tc_matmul_stats(x, W, bn=1000)
    table = xw.reshape(R * N, D)

    edata = jnp.stack([edge_index[0].astype(jnp.float32),
                       rel_type.astype(jnp.float32),
                       edge_index[1].astype(jnp.float32), norm], axis=1)
    if e_pad:
        edata = jnp.concatenate(
            [edata, jnp.zeros((e_pad, 4), jnp.float32)], axis=0)
    edata = edata.reshape(NW, G, C, 4).transpose(0, 1, 3, 2)

    zeros = jnp.zeros((NPAD, D), jnp.float32)
    partials = _sc_edges(table, edata, zeros, N, NPAD, D, G)

    return _tc_combine(x, partials, stats, gamma.reshape(1, D),
                       beta.reshape(1, D), bn=1000)
